# Initial kernel scaffold; baseline (speedup 1.0000x reference)
import jax
import jax.numpy as jnp
from jax.experimental import pallas as pl


def kernel(X, topk_indices, class_to_task, weight, sigma):
    raise NotImplementedError("write your pallas kernel here")


# speedup vs baseline: 3.5751x; 3.5751x over previous
import functools

import jax
import jax.numpy as jnp
from jax.experimental import pallas as pl


def _topk_cosine_kernel(x_ref, w_ref, ctt_ref, topk_ref, out_ref, *, T, C, K):
    BN = x_ref.shape[1]

    w = w_ref[...]
    wnorm = jnp.sqrt(jnp.sum(w * w, axis=1, keepdims=True))
    wn = w * (1.0 / jnp.maximum(wnorm, 1e-12))

    ctt = ctt_ref[0, :]

    acc = jnp.zeros((BN, C), jnp.float32)
    for t in range(T):
        xt = x_ref[t]
        xn_norm = jnp.sqrt(jnp.sum(xt * xt, axis=1, keepdims=True))
        xn = xt * (1.0 / jnp.maximum(xn_norm, 1e-12))
        mt = jax.lax.dot_general(
            xn, wn, (((1,), (1,)), ((), ())),
            precision=jax.lax.Precision.HIGHEST,
            preferred_element_type=jnp.float32,
        )
        acc = jnp.where((ctt == t)[None, :], mt, acc)

    lane = jax.lax.broadcasted_iota(jnp.int32, (BN, C), 1)
    neg_inf = jnp.float32(-jnp.inf)
    best_val = jnp.full((BN, 1), neg_inf, jnp.float32)
    best_lbl = jnp.zeros((BN, 1), jnp.int32)
    for k in range(K):
        ck = topk_ref[:, k][:, None]
        val_k = jnp.max(jnp.where(lane == ck, acc, neg_inf), axis=1,
                        keepdims=True)
        better = val_k > best_val
        best_val = jnp.where(better, val_k, best_val)
        best_lbl = jnp.where(better, ck, best_lbl)
    out_ref[...] = best_lbl


def kernel(X, topk_indices, class_to_task, weight, sigma):
    del sigma
    T, N, D = X.shape
    C = weight.shape[0]
    K = topk_indices.shape[1]
    BN = 256

    ctt2d = class_to_task.reshape(1, C).astype(jnp.int32)
    topk = topk_indices.astype(jnp.int32)

    out = pl.pallas_call(
        functools.partial(_topk_cosine_kernel, T=T, C=C, K=K),
        grid=(N // BN,),
        in_specs=[
            pl.BlockSpec((T, BN, D), lambda i: (0, i, 0)),
            pl.BlockSpec((C, D), lambda i: (0, 0)),
            pl.BlockSpec((1, C), lambda i: (0, 0)),
            pl.BlockSpec((BN, K), lambda i: (i, 0)),
        ],
        out_specs=pl.BlockSpec((BN, 1), lambda i: (i, 0)),
        out_shape=jax.ShapeDtypeStruct((N, 1), jnp.int32),
    )(X, weight, ctt2d, topk)
    return out[:, 0]


# speedup vs baseline: 3.9409x; 1.1023x over previous
import functools

import jax
import jax.numpy as jnp
from jax import lax
from jax.experimental import pallas as pl
from jax.experimental.pallas import tpu as pltpu
from jax.experimental.pallas import tpu_sc as plsc


def _sim_table_kernel(x_ref, w_ref, ctt_ref, out_ref, *, T, C, CP):
    BN = x_ref.shape[1]

    w = w_ref[...]
    wnorm = jnp.sqrt(jnp.sum(w * w, axis=1, keepdims=True))
    wn = w * (1.0 / jnp.maximum(wnorm, 1e-12))

    ctt = ctt_ref[0, :]

    acc = jnp.zeros((BN, C), jnp.float32)
    for t in range(T):
        xt = x_ref[t]
        xn_norm = jnp.sqrt(jnp.sum(xt * xt, axis=1, keepdims=True))
        xn = xt * (1.0 / jnp.maximum(xn_norm, 1e-12))
        mt = jax.lax.dot_general(
            xn, wn, (((1,), (1,)), ((), ())),
            precision=jax.lax.Precision.HIGHEST,
            preferred_element_type=jnp.float32,
        )
        acc = jnp.where((ctt == t)[None, :], mt, acc)

    out_ref[...] = jnp.concatenate(
        [acc, jnp.zeros((BN, CP - C), jnp.float32)], axis=1)


def _make_select_kernel(N, CP, K, NW):
    spw = N // NW
    n_groups = spw // 16
    mesh = plsc.VectorSubcoreMesh(core_axis_name="c", subcore_axis_name="s")

    @functools.partial(
        pl.kernel, mesh=mesh,
        out_type=jax.ShapeDtypeStruct((N,), jnp.int32),
        compiler_params=pltpu.CompilerParams(
            use_tc_tiling_on_sc=False, needs_layout_passes=False),
        scratch_types=[
            pltpu.VMEM((K * spw,), jnp.int32),
            pltpu.VMEM((16 * CP,), jnp.float32),
            pltpu.VMEM((spw,), jnp.int32),
        ],
    )
    def sel(a_hbm, topkw_hbm, out_hbm, topkw_v, rows_v, lbl_v):
        nc = 2
        wid = lax.axis_index("s") * nc + lax.axis_index("c")
        base = wid * spw
        pltpu.sync_copy(topkw_hbm.at[pl.ds(wid * K * spw, K * spw)], topkw_v)
        row_base = lax.iota(jnp.int32, 16) * CP

        def group_body(g, carry):
            n0 = base + g * 16
            pltpu.sync_copy(a_hbm.at[pl.ds(n0 * CP, 16 * CP)], rows_v)
            best_val = jnp.full((16,), -jnp.inf, jnp.float32)
            best_lbl = jnp.zeros((16,), jnp.int32)
            for k in range(K):
                c_vec = topkw_v[pl.ds(k * spw + g * 16, 16)]
                vals = plsc.load_gather(rows_v, [row_base + c_vec])
                better = vals > best_val
                best_val = jnp.where(better, vals, best_val)
                best_lbl = jnp.where(better, c_vec, best_lbl)
            lbl_v[pl.ds(g * 16, 16)] = best_lbl
            return carry

        lax.fori_loop(0, n_groups, group_body, 0)
        pltpu.sync_copy(lbl_v, out_hbm.at[pl.ds(base, spw)])

    return sel


def kernel(X, topk_indices, class_to_task, weight, sigma):
    del sigma
    T, N, D = X.shape
    C = weight.shape[0]
    K = topk_indices.shape[1]
    BN = 256
    CP = 1024
    NW = 32

    spw = N // NW
    ctt2d = class_to_task.reshape(1, C).astype(jnp.int32)
    topk_w = (topk_indices.astype(jnp.int32).T
              .reshape(K, NW, spw).transpose(1, 0, 2).reshape(-1))

    a = pl.pallas_call(
        functools.partial(_sim_table_kernel, T=T, C=C, CP=CP),
        grid=(N // BN,),
        in_specs=[
            pl.BlockSpec((T, BN, D), lambda i: (0, i, 0)),
            pl.BlockSpec((C, D), lambda i: (0, 0)),
            pl.BlockSpec((1, C), lambda i: (0, 0)),
        ],
        out_specs=pl.BlockSpec((BN, CP), lambda i: (i, 0)),
        out_shape=jax.ShapeDtypeStruct((N, CP), jnp.float32),
    )(X, weight, ctt2d)

    return _make_select_kernel(N, CP, K, NW)(a.reshape(-1), topk_w)


# speedup vs baseline: 4.4992x; 1.1417x over previous
import functools

import jax
import jax.numpy as jnp
from jax import lax
from jax.experimental import pallas as pl
from jax.experimental.pallas import tpu as pltpu
from jax.experimental.pallas import tpu_sc as plsc

_SC_PARAMS = pltpu.CompilerParams(
    use_tc_tiling_on_sc=False, needs_layout_passes=False)


def _make_reorder_kernel(ROWS, D, NW):
    rpw = ROWS // NW
    mesh = plsc.VectorSubcoreMesh(core_axis_name="c", subcore_axis_name="s")

    @functools.partial(
        pl.kernel, mesh=mesh,
        out_type=jax.ShapeDtypeStruct((ROWS, D), jnp.float32),
        compiler_params=_SC_PARAMS,
        scratch_types=[
            pltpu.VMEM((rpw,), jnp.int32),
            pltpu.VMEM((rpw, D), jnp.float32),
            pltpu.SemaphoreType.DMA,
        ],
    )
    def reorder(w_hbm, idx_hbm, out_hbm, idx_v, rows_v, sem):
        wid = lax.axis_index("s") * 2 + lax.axis_index("c")
        base = wid * rpw
        pltpu.sync_copy(idx_hbm.at[pl.ds(base, rpw)], idx_v)
        pltpu.async_copy(w_hbm.at[idx_v], rows_v, sem).wait()
        pltpu.sync_copy(rows_v, out_hbm.at[pl.ds(base, rpw), :])

    return reorder


def _sim_sorted_kernel(task_ref, x_ref, ws_ref, out_ref, inv_ref, wn_ref, *,
                       T, NCHUNK, CW):
    BN = x_ref.shape[1]

    @pl.when(pl.program_id(0) == 0)
    def _():
        for j in range(NCHUNK):
            w = ws_ref[pl.ds(j * CW, CW), :]
            wn = jnp.sqrt(jnp.sum(w * w, axis=1, keepdims=True))
            wn_ref[pl.ds(j * CW, CW), :] = w * (1.0 / jnp.maximum(wn, 1e-12))

    for t in range(T):
        xt = x_ref[t]
        ss = jnp.sqrt(jnp.sum(xt * xt, axis=1))
        inv_ref[t, :] = 1.0 / jnp.maximum(ss, 1e-12)

    for j in range(NCHUNK):
        tj = task_ref[j]

        @pl.when(tj >= 0)
        def _():
            xt = x_ref[tj]
            wn = wn_ref[pl.ds(j * CW, CW), :]
            mt = lax.dot_general(
                xt, wn, (((1,), (1,)), ((), ())),
                precision=lax.Precision.HIGHEST,
                preferred_element_type=jnp.float32,
            )
            out_ref[:, j * CW:(j + 1) * CW] = mt * inv_ref[tj, :][:, None]

        @pl.when(tj < 0)
        def _():
            out_ref[:, j * CW:(j + 1) * CW] = jnp.zeros((BN, CW), jnp.float32)


def _make_select_kernel(N, CP, CPAD, K, NW):
    spw = N // NW
    n_groups = spw // 16
    mesh = plsc.VectorSubcoreMesh(core_axis_name="c", subcore_axis_name="s")

    @functools.partial(
        pl.kernel, mesh=mesh,
        out_type=jax.ShapeDtypeStruct((N,), jnp.int32),
        compiler_params=_SC_PARAMS,
        scratch_types=[
            pltpu.VMEM((K * spw,), jnp.int32),
            pltpu.VMEM((CPAD,), jnp.int32),
            pltpu.VMEM((16 * CP,), jnp.float32),
            pltpu.VMEM((spw,), jnp.int32),
        ],
    )
    def sel(a_hbm, topkw_hbm, pos_hbm, out_hbm, topkw_v, pos_v, rows_v, lbl_v):
        wid = lax.axis_index("s") * 2 + lax.axis_index("c")
        base = wid * spw
        pltpu.sync_copy(topkw_hbm.at[pl.ds(wid * K * spw, K * spw)], topkw_v)
        pltpu.sync_copy(pos_hbm, pos_v)
        row_base = lax.iota(jnp.int32, 16) * CP

        def group_body(g, carry):
            n0 = base + g * 16
            pltpu.sync_copy(a_hbm.at[pl.ds(n0 * CP, 16 * CP)], rows_v)
            best_val = jnp.full((16,), -jnp.inf, jnp.float32)
            best_lbl = jnp.zeros((16,), jnp.int32)
            for k in range(K):
                c_vec = topkw_v[pl.ds(k * spw + g * 16, 16)]
                p_vec = plsc.load_gather(pos_v, [c_vec])
                vals = plsc.load_gather(rows_v, [row_base + p_vec])
                better = vals > best_val
                best_val = jnp.where(better, vals, best_val)
                best_lbl = jnp.where(better, c_vec, best_lbl)
            lbl_v[pl.ds(g * 16, 16)] = best_lbl
            return carry

        lax.fori_loop(0, n_groups, group_body, 0)
        pltpu.sync_copy(lbl_v, out_hbm.at[pl.ds(base, spw)])

    return sel


def _schedule(class_to_task, T, C, CW, NCHUNK, CPAD):
    ctt = class_to_task.astype(jnp.int32)
    order = jnp.argsort(ctt)
    counts = jnp.zeros((T,), jnp.int32).at[ctt].add(1)
    nchunks = (counts + CW - 1) // CW
    cum = jnp.cumsum(nchunks)
    first_chunk = cum - nchunks
    total_chunks = cum[-1]
    jidx = jnp.arange(NCHUNK, dtype=jnp.int32)
    chunk_task = jnp.searchsorted(cum, jidx, side="right").astype(jnp.int32)
    chunk_task = jnp.where(jidx < total_chunks, chunk_task, -1)
    group_start = jnp.cumsum(counts) - counts

    rank = jnp.arange(C, dtype=jnp.int32)
    ctt_sorted = ctt[order]
    slot = first_chunk[ctt_sorted] * CW + (rank - group_start[ctt_sorted])
    pos = jnp.zeros((C,), jnp.int32).at[order].set(slot)
    pos_pad = jnp.concatenate([pos, jnp.zeros((CPAD - C,), jnp.int32)])

    s = jnp.arange(NCHUNK * CW, dtype=jnp.int32)
    j_of_s = s // CW
    t_of_s = chunk_task[j_of_s]
    t_cl = jnp.maximum(t_of_s, 0)
    r = (j_of_s - first_chunk[t_cl]) * CW + (s % CW)
    valid = (t_of_s >= 0) & (r < counts[t_cl])
    src = order[jnp.clip(group_start[t_cl] + r, 0, C - 1)]
    order_pad = jnp.where(valid, src, 0).astype(jnp.int32)
    return chunk_task, pos_pad, order_pad


def kernel(X, topk_indices, class_to_task, weight, sigma):
    del sigma
    T, N, D = X.shape
    C = weight.shape[0]
    K = topk_indices.shape[1]
    BN = 256
    CW = 128
    NCHUNK = 20
    CP = NCHUNK * CW
    CPAD = 1024
    NW = 32
    spw = N // NW

    chunk_task, pos_pad, order_pad = _schedule(
        class_to_task, T, C, CW, NCHUNK, CPAD)

    topk_w = (topk_indices.astype(jnp.int32).T
              .reshape(K, NW, spw).transpose(1, 0, 2).reshape(-1))

    ws = _make_reorder_kernel(CP, D, NW)(weight, order_pad)

    a = pl.pallas_call(
        functools.partial(_sim_sorted_kernel, T=T, NCHUNK=NCHUNK, CW=CW),
        grid=(N // BN,),
        in_specs=[
            pl.BlockSpec(memory_space=pltpu.SMEM),
            pl.BlockSpec((T, BN, D), lambda i: (0, i, 0)),
            pl.BlockSpec((CP, D), lambda i: (0, 0)),
        ],
        out_specs=pl.BlockSpec((BN, CP), lambda i: (i, 0)),
        out_shape=jax.ShapeDtypeStruct((N, CP), jnp.float32),
        scratch_shapes=[
            pltpu.VMEM((T, BN), jnp.float32),
            pltpu.VMEM((CP, D), jnp.float32),
        ],
    )(chunk_task, X, ws)

    return _make_select_kernel(N, CP, CPAD, K, NW)(
        a.reshape(-1), topk_w, pos_pad)


# speedup vs baseline: 4.5428x; 1.0097x over previous
import functools

import jax
import jax.numpy as jnp
from jax import lax
from jax.experimental import pallas as pl
from jax.experimental.pallas import tpu as pltpu
from jax.experimental.pallas import tpu_sc as plsc

_SC_PARAMS = pltpu.CompilerParams(
    use_tc_tiling_on_sc=False, needs_layout_passes=False)


def _make_reorder_kernel(ROWS, D, NW):
    rpw = ROWS // NW
    mesh = plsc.VectorSubcoreMesh(core_axis_name="c", subcore_axis_name="s")

    @functools.partial(
        pl.kernel, mesh=mesh,
        out_type=jax.ShapeDtypeStruct((ROWS, D), jnp.float32),
        compiler_params=_SC_PARAMS,
        scratch_types=[
            pltpu.VMEM((rpw,), jnp.int32),
            pltpu.VMEM((rpw, D), jnp.float32),
            pltpu.SemaphoreType.DMA,
        ],
    )
    def reorder(w_hbm, idx_hbm, out_hbm, idx_v, rows_v, sem):
        wid = lax.axis_index("s") * 2 + lax.axis_index("c")
        base = wid * rpw
        pltpu.sync_copy(idx_hbm.at[pl.ds(base, rpw)], idx_v)
        pltpu.async_copy(w_hbm.at[idx_v], rows_v, sem).wait()
        pltpu.sync_copy(rows_v, out_hbm.at[pl.ds(base, rpw), :])

    return reorder


def _sim_sorted_kernel(task_ref, x_ref, ws_ref, out_ref, inv_ref, wn_ref, *,
                       T, NCHUNK, CW):
    @pl.when(pl.program_id(0) == 0)
    def _():
        for j in range(NCHUNK):
            w = ws_ref[pl.ds(j * CW, CW), :]
            wn = jnp.sqrt(jnp.sum(w * w, axis=1, keepdims=True))
            wn_ref[pl.ds(j * CW, CW), :] = w * (1.0 / jnp.maximum(wn, 1e-12))

    for t in range(T):
        xt = x_ref[t]
        ss = jnp.sqrt(jnp.sum(xt * xt, axis=1))
        inv_ref[t, :] = 1.0 / jnp.maximum(ss, 1e-12)

    def chunk_body(j, carry):
        tj = task_ref[j]
        xt = x_ref[tj]
        wn = wn_ref[pl.ds(j * CW, CW), :]
        mt = lax.dot_general(
            wn, xt, (((1,), (1,)), ((), ())),
            precision=lax.Precision.HIGHEST,
            preferred_element_type=jnp.float32,
        )
        out_ref[pl.ds(j * CW, CW), :] = mt * inv_ref[tj, :][None, :]
        return carry

    lax.fori_loop(0, task_ref[NCHUNK], chunk_body, 0)


def _make_select_kernel(N, CP, CPAD, K, NW):
    spw = N // NW
    nv = (K * spw) // 16
    n_groups = spw // 16
    GC = 128
    mesh = plsc.VectorSubcoreMesh(core_axis_name="c", subcore_axis_name="s")

    @functools.partial(
        pl.kernel, mesh=mesh,
        out_type=jax.ShapeDtypeStruct((N,), jnp.int32),
        compiler_params=_SC_PARAMS,
        scratch_types=[
            pltpu.VMEM((K * spw,), jnp.int32),
            pltpu.VMEM((CPAD,), jnp.int32),
            pltpu.VMEM((K * spw,), jnp.int32),
            pltpu.VMEM((K * spw,), jnp.float32),
            pltpu.VMEM((spw,), jnp.int32),
            pltpu.SemaphoreType.DMA,
        ],
    )
    def sel(a_hbm, topkw_hbm, pos_hbm, out_hbm, topkw_v, pos_v, idx_v, vals_v,
            lbl_v, sem):
        wid = lax.axis_index("s") * 2 + lax.axis_index("c")
        base = wid * spw
        pltpu.sync_copy(topkw_hbm.at[pl.ds(wid * K * spw, K * spw)], topkw_v)
        pltpu.sync_copy(pos_hbm, pos_v)
        lane = lax.iota(jnp.int32, 16)

        def idx_body(v, carry):
            c_vec = topkw_v[pl.ds(v * 16, 16)]
            p_vec = plsc.load_gather(pos_v, [c_vec])
            nl = jnp.bitwise_and(v * 16 + lane, spw - 1)
            idx_v[pl.ds(v * 16, 16)] = p_vec * N + (base + nl)
            return carry

        lax.fori_loop(0, nv, idx_body, 0)

        copies = []
        for b in range((K * spw) // GC):
            copies.append(pltpu.make_async_copy(
                a_hbm.at[idx_v.at[pl.ds(b * GC, GC)]],
                vals_v.at[pl.ds(b * GC, GC)], sem))
        for cp in copies:
            cp.start()
        for cp in copies:
            cp.wait()

        def group_body(g, carry):
            best_val = jnp.full((16,), -jnp.inf, jnp.float32)
            best_lbl = jnp.zeros((16,), jnp.int32)
            for k in range(K):
                off = k * spw + g * 16
                c_vec = topkw_v[pl.ds(off, 16)]
                vals = vals_v[pl.ds(off, 16)]
                better = vals > best_val
                best_val = jnp.where(better, vals, best_val)
                best_lbl = jnp.where(better, c_vec, best_lbl)
            lbl_v[pl.ds(g * 16, 16)] = best_lbl
            return carry

        lax.fori_loop(0, n_groups, group_body, 0)
        pltpu.sync_copy(lbl_v, out_hbm.at[pl.ds(base, spw)])

    return sel


def _schedule(class_to_task, T, C, CW, NCHUNK, CPAD):
    ctt = class_to_task.astype(jnp.int32)
    order = jnp.argsort(ctt)
    counts = jnp.zeros((T,), jnp.int32).at[ctt].add(1)
    nchunks = (counts + CW - 1) // CW
    cum = jnp.cumsum(nchunks)
    first_chunk = cum - nchunks
    total_chunks = cum[-1]
    jidx = jnp.arange(NCHUNK, dtype=jnp.int32)
    chunk_task = jnp.searchsorted(cum, jidx, side="right").astype(jnp.int32)
    chunk_task = jnp.where(jidx < total_chunks, chunk_task, -1)
    chunk_task = jnp.concatenate(
        [chunk_task, total_chunks[None].astype(jnp.int32)])
    group_start = jnp.cumsum(counts) - counts

    rank = jnp.arange(C, dtype=jnp.int32)
    ctt_sorted = ctt[order]
    slot = first_chunk[ctt_sorted] * CW + (rank - group_start[ctt_sorted])
    pos = jnp.zeros((C,), jnp.int32).at[order].set(slot)
    pos_pad = jnp.concatenate([pos, jnp.zeros((CPAD - C,), jnp.int32)])

    s = jnp.arange(NCHUNK * CW, dtype=jnp.int32)
    j_of_s = s // CW
    t_of_s = chunk_task[j_of_s]
    t_cl = jnp.maximum(t_of_s, 0)
    r = (j_of_s - first_chunk[t_cl]) * CW + (s % CW)
    valid = (t_of_s >= 0) & (r < counts[t_cl])
    src = order[jnp.clip(group_start[t_cl] + r, 0, C - 1)]
    order_pad = jnp.where(valid, src, 0).astype(jnp.int32)
    return chunk_task, pos_pad, order_pad


def kernel(X, topk_indices, class_to_task, weight, sigma):
    del sigma
    T, N, D = X.shape
    C = weight.shape[0]
    K = topk_indices.shape[1]
    BN = 256
    CW = 128
    NCHUNK = 20
    CP = NCHUNK * CW
    CPAD = 1024
    NW = 32
    spw = N // NW

    chunk_task, pos_pad, order_pad = _schedule(
        class_to_task, T, C, CW, NCHUNK, CPAD)

    topk_w = (topk_indices.astype(jnp.int32).T
              .reshape(K, NW, spw).transpose(1, 0, 2).reshape(-1))

    ws = _make_reorder_kernel(CP, D, NW)(weight, order_pad)

    a = pl.pallas_call(
        functools.partial(_sim_sorted_kernel, T=T, NCHUNK=NCHUNK, CW=CW),
        grid=(N // BN,),
        in_specs=[
            pl.BlockSpec(memory_space=pltpu.SMEM),
            pl.BlockSpec((T, BN, D), lambda i: (0, i, 0)),
            pl.BlockSpec((CP, D), lambda i: (0, 0)),
        ],
        out_specs=pl.BlockSpec((CP, BN), lambda i: (0, i)),
        out_shape=jax.ShapeDtypeStruct((CP, N), jnp.float32),
        scratch_shapes=[
            pltpu.VMEM((T, BN), jnp.float32),
            pltpu.VMEM((CP, D), jnp.float32),
        ],
    )(chunk_task, X, ws)

    return _make_select_kernel(N, CP, CPAD, K, NW)(
        a.reshape(-1), topk_w, pos_pad)


# speedup vs baseline: 5.4295x; 1.1952x over previous
import functools

import jax
import jax.numpy as jnp
from jax import lax
from jax.experimental import pallas as pl
from jax.experimental.pallas import tpu as pltpu
from jax.experimental.pallas import tpu_sc as plsc

_SC_PARAMS = pltpu.CompilerParams(
    use_tc_tiling_on_sc=False, needs_layout_passes=False)


def _sim_sorted_kernel(task_ref, x_ref, w_ref, op_ref, out_ref, inv_ref,
                       wn_ref, *, T, NCHUNK, CW, C):
    @pl.when(pl.program_id(0) == 0)
    def _():
        cn = w_ref.shape[0]
        cls = lax.broadcasted_iota(jnp.int32, (CW, cn), 1)

        def perm_body(j, carry):
            opj = op_ref[pl.ds(j * CW, CW), :]
            p = (opj == cls).astype(jnp.float32)
            ws = lax.dot_general(
                p, w_ref[...], (((1,), (0,)), ((), ())),
                precision=lax.Precision.HIGHEST,
                preferred_element_type=jnp.float32,
            )
            wn = jnp.sqrt(jnp.sum(ws * ws, axis=1, keepdims=True))
            wn_ref[pl.ds(j * CW, CW), :] = ws * (1.0 / jnp.maximum(wn, 1e-12))
            return carry

        lax.fori_loop(0, task_ref[NCHUNK], perm_body, 0)

    for t in range(T):
        xt = x_ref[t]
        ss = jnp.sqrt(jnp.sum(xt * xt, axis=1))
        inv_ref[t, :] = 1.0 / jnp.maximum(ss, 1e-12)

    def chunk_body(j, carry):
        tj = task_ref[j]
        xt = x_ref[tj]
        wn = wn_ref[pl.ds(j * CW, CW), :]
        mt = lax.dot_general(
            wn, xt, (((1,), (1,)), ((), ())),
            precision=lax.Precision.HIGHEST,
            preferred_element_type=jnp.float32,
        )
        out_ref[pl.ds(j * CW, CW), :] = mt * inv_ref[tj, :][None, :]
        return carry

    lax.fori_loop(0, task_ref[NCHUNK], chunk_body, 0)


def _make_select_kernel(N, CP, CPAD, K, NW):
    spw = N // NW
    nv = (K * spw) // 16
    n_groups = spw // 16
    GC = 128
    mesh = plsc.VectorSubcoreMesh(core_axis_name="c", subcore_axis_name="s")

    @functools.partial(
        pl.kernel, mesh=mesh,
        out_type=jax.ShapeDtypeStruct((N,), jnp.int32),
        compiler_params=_SC_PARAMS,
        scratch_types=[
            pltpu.VMEM((K * spw,), jnp.int32),
            pltpu.VMEM((K * spw,), jnp.int32),
            pltpu.VMEM((CPAD,), jnp.int32),
            pltpu.VMEM((K * spw,), jnp.int32),
            pltpu.VMEM((K * spw,), jnp.float32),
            pltpu.VMEM((spw,), jnp.int32),
            pltpu.SemaphoreType.DMA,
        ],
    )
    def sel(a_hbm, topk_hbm, pos_hbm, out_hbm, topkn_v, topkw_v, pos_v, idx_v,
            vals_v, lbl_v, sem):
        wid = lax.axis_index("s") * 2 + lax.axis_index("c")
        base = wid * spw
        pltpu.sync_copy(topk_hbm.at[pl.ds(base * K, spw * K)], topkn_v)
        pltpu.sync_copy(pos_hbm, pos_v)
        lane = lax.iota(jnp.int32, 16)
        vps = spw // 16

        def idx_body(v, carry):
            k = v // vps
            nl = (v % vps) * 16 + lane
            c_vec = plsc.load_gather(topkn_v, [nl * K + k])
            topkw_v[pl.ds(v * 16, 16)] = c_vec
            p_vec = plsc.load_gather(pos_v, [c_vec])
            idx_v[pl.ds(v * 16, 16)] = p_vec * N + (base + nl)
            return carry

        lax.fori_loop(0, nv, idx_body, 0)

        copies = []
        for b in range((K * spw) // GC):
            copies.append(pltpu.make_async_copy(
                a_hbm.at[idx_v.at[pl.ds(b * GC, GC)]],
                vals_v.at[pl.ds(b * GC, GC)], sem))
        for cp in copies:
            cp.start()
        for cp in copies:
            cp.wait()

        def group_body(g, carry):
            best_val = jnp.full((16,), -jnp.inf, jnp.float32)
            best_lbl = jnp.zeros((16,), jnp.int32)
            for k in range(K):
                off = k * spw + g * 16
                c_vec = topkw_v[pl.ds(off, 16)]
                vals = vals_v[pl.ds(off, 16)]
                better = vals > best_val
                best_val = jnp.where(better, vals, best_val)
                best_lbl = jnp.where(better, c_vec, best_lbl)
            lbl_v[pl.ds(g * 16, 16)] = best_lbl
            return carry

        lax.fori_loop(0, n_groups, group_body, 0)
        pltpu.sync_copy(lbl_v, out_hbm.at[pl.ds(base, spw)])

    return sel


def _schedule(class_to_task, T, C, CW, NCHUNK, CPAD):
    ctt = class_to_task.astype(jnp.int32)
    order = jnp.argsort(ctt)
    counts = jnp.zeros((T,), jnp.int32).at[ctt].add(1)
    nchunks = (counts + CW - 1) // CW
    cum = jnp.cumsum(nchunks)
    first_chunk = cum - nchunks
    total_chunks = cum[-1]
    jidx = jnp.arange(NCHUNK, dtype=jnp.int32)
    chunk_task = jnp.searchsorted(cum, jidx, side="right").astype(jnp.int32)
    chunk_task = jnp.where(jidx < total_chunks, chunk_task, -1)
    chunk_task = jnp.concatenate(
        [chunk_task, total_chunks[None].astype(jnp.int32)])
    group_start = jnp.cumsum(counts) - counts

    rank = jnp.arange(C, dtype=jnp.int32)
    ctt_sorted = ctt[order]
    slot = first_chunk[ctt_sorted] * CW + (rank - group_start[ctt_sorted])
    pos = jnp.zeros((C,), jnp.int32).at[order].set(slot)
    pos_pad = jnp.concatenate([pos, jnp.zeros((CPAD - C,), jnp.int32)])

    s = jnp.arange(NCHUNK * CW, dtype=jnp.int32)
    j_of_s = s // CW
    t_of_s = chunk_task[j_of_s]
    t_cl = jnp.maximum(t_of_s, 0)
    r = (j_of_s - first_chunk[t_cl]) * CW + (s % CW)
    valid = (t_of_s >= 0) & (r < counts[t_cl])
    src = order[jnp.clip(group_start[t_cl] + r, 0, C - 1)]
    order_pad = jnp.where(valid, src, 0).astype(jnp.int32)
    return chunk_task, pos_pad, order_pad


def kernel(X, topk_indices, class_to_task, weight, sigma):
    del sigma
    T, N, D = X.shape
    C = weight.shape[0]
    K = topk_indices.shape[1]
    BN = 256
    CW = 128
    NCHUNK = 20
    CP = NCHUNK * CW
    CPAD = 1024
    NW = 32
    spw = N // NW

    chunk_task, pos_pad, order_pad = _schedule(
        class_to_task, T, C, CW, NCHUNK, CPAD)

    a = pl.pallas_call(
        functools.partial(_sim_sorted_kernel, T=T, NCHUNK=NCHUNK, CW=CW, C=C),
        grid=(N // BN,),
        in_specs=[
            pl.BlockSpec(memory_space=pltpu.SMEM),
            pl.BlockSpec((T, BN, D), lambda i: (0, i, 0)),
            pl.BlockSpec((C, D), lambda i: (0, 0)),
            pl.BlockSpec((CP, 1), lambda i: (0, 0)),
        ],
        out_specs=pl.BlockSpec((CP, BN), lambda i: (0, i)),
        out_shape=jax.ShapeDtypeStruct((CP, N), jnp.float32),
        scratch_shapes=[
            pltpu.VMEM((T, BN), jnp.float32),
            pltpu.VMEM((CP, D), jnp.float32),
        ],
    )(chunk_task, X, weight, order_pad.reshape(CP, 1))

    return _make_select_kernel(N, CP, CPAD, K, NW)(
        a.reshape(-1), topk_indices.astype(jnp.int32).reshape(-1), pos_pad)
